# baseline (speedup 1.0000x reference)
import functools

import jax
import jax.numpy as jnp
from jax import lax
from jax.experimental import pallas as pl
from jax.experimental.pallas import tpu as pltpu
from jax.experimental.pallas import tpu_sc as plsc

NC, NS, L = 2, 16, 16
CHUNK = 128
N_FIX = 10000
E_FIX = 160000
E_PAD = 163840
NCHUNKS = E_PAD // CHUNK
ROWS_PER_TILE = N_FIX // NS
ACC_ROWS = N_FIX + 8
CP = 128


def _sc_mesh():
  return plsc.VectorSubcoreMesh(core_axis_name="c", subcore_axis_name="s")


def _tile_rows(s):
  r0 = pl.multiple_of(s * 624, 8)
  return r0


def _row_copy(s, mk_src, mk_dst):
  @pl.when(s < NS - 1)
  def _():
    r0 = _tile_rows(s)
    pltpu.sync_copy(mk_src(r0, 624), mk_dst(r0, 624))
  @pl.when(s == NS - 1)
  def _():
    pltpu.sync_copy(mk_src(624 * (NS - 1), 640), mk_dst(624 * (NS - 1), 640))


def _deg_body(dstp, out, idx_v, hist, sem):
  c = lax.axis_index("c")
  s = lax.axis_index("s")
  wid = c * NS + s
  chunks_per_tile = NCHUNKS // (NC * NS)

  def zero_body(i, carry):
    hist[pl.ds(i * L, L)] = jnp.zeros((L,), jnp.float32)
    return carry
  lax.fori_loop(0, 10240 // L, zero_body, 0)

  ones = jnp.ones((L,), jnp.float32)

  def body(j, carry):
    pltpu.sync_copy(dstp.at[wid * chunks_per_tile + j], idx_v)
    for k in range(CHUNK // L):
      idx = idx_v[pl.ds(k * L, L)]
      plsc.addupdate_scatter(hist, [idx], ones)
    return carry
  lax.fori_loop(0, chunks_per_tile, body, 0)

  pltpu.sync_copy(hist, out.at[wid])


def _sc_degree(dstp):
  kern = pl.kernel(
      _deg_body,
      out_type=jax.ShapeDtypeStruct((NC * NS, 10240), jnp.float32),
      mesh=_sc_mesh(),
      compiler_params=pltpu.CompilerParams(needs_layout_passes=False),
      scratch_types=[
          pltpu.VMEM((CHUNK,), jnp.int32),
          pltpu.VMEM((10240,), jnp.float32),
          pltpu.SemaphoreType.DMA,
      ],
  )
  return kern(dstp)


NBUF = 2
IDXB = 40


def _edge_pipeline(load_idx, gather_issue, gather_wait, scatter_issue,
                   scatter_wait, nphases):
  for p in range(nphases):
    load_idx(p)
    for b in range(NBUF):
      gather_issue(b, b)

    def body(t, carry):
      for b in range(NBUF):
        j = t * NBUF + b
        gather_wait(j, b)
        scatter_issue(j, b)
      for b in range(NBUF):
        j = t * NBUF + b
        @pl.when(j + NBUF < IDXB)
        def _(j=j, b=b):
          scatter_wait(j, b)
          gather_issue(j + NBUF, b)
      return carry
    lax.fori_loop(0, IDXB // NBUF, body, 0)
    for b in range(NBUF):
      scatter_wait(IDXB - NBUF + b, b)


def _spmm_body(g0, g1, srcp, dstp, out, src2d, dst2d, b0, b1,
               acc, g0s, g1s, s0s, s1s):
  c = lax.axis_index("c")
  s = lax.axis_index("s")
  bufs = (b0, b1)
  gsems = (g0s, g1s)
  ssems = (s0s, s1s)

  chunks_per_tile = NCHUNKS // NS
  for ci in (0, 1):
    @pl.when(c == ci)
    def _(ci=ci):
      g_c = (g0, g1)[ci]
      _row_copy(s, lambda o, sz: g_c.at[pl.ds(o, sz)],
                lambda o, sz: acc.at[pl.ds(o, sz)])
  plsc.subcore_barrier()

  row0 = s * chunks_per_tile

  def load_idx(p):
    pltpu.sync_copy(srcp.at[pl.ds(row0 + p * IDXB, IDXB)], src2d)
    pltpu.sync_copy(dstp.at[pl.ds(row0 + p * IDXB, IDXB)], dst2d)

  def gather_issue(j, b):
    for ci in (0, 1):
      @pl.when(c == ci)
      def _(ci=ci):
        pltpu.async_copy((g0, g1)[ci].at[src2d.at[j]], bufs[b], gsems[b])

  def gather_wait(j, b):
    for ci in (0, 1):
      @pl.when(c == ci)
      def _(ci=ci):
        pltpu.make_async_copy((g0, g1)[ci].at[src2d.at[j]], bufs[b],
                              gsems[b]).wait()

  def scatter_issue(j, b):
    pltpu.async_copy(bufs[b], acc.at[dst2d.at[j]], ssems[b], add=True)

  def scatter_wait(j, b):
    pltpu.make_async_copy(bufs[b], acc.at[dst2d.at[j]], ssems[b]).wait()

  _edge_pipeline(load_idx, gather_issue, gather_wait, scatter_issue,
                 scatter_wait, chunks_per_tile // IDXB)

  plsc.subcore_barrier()
  for ci in (0, 1):
    @pl.when(c == ci)
    def _(ci=ci):
      _row_copy(s, lambda o, sz: acc.at[pl.ds(o, sz)],
                lambda o, sz: out.at[ci, pl.ds(o, sz)])


def _sc_spmm256(g0, g1, srcp, dstp):
  kern = pl.kernel(
      _spmm_body,
      out_type=jax.ShapeDtypeStruct((NC, N_FIX, 128), jnp.float32),
      mesh=_sc_mesh(),
      compiler_params=pltpu.CompilerParams(needs_layout_passes=False,
                                           use_tc_tiling_on_sc=False),
      scratch_types=[
          pltpu.VMEM((IDXB, CHUNK), jnp.int32),
          pltpu.VMEM((IDXB, CHUNK), jnp.int32),
      ] + [pltpu.VMEM((CHUNK, 128), jnp.float32)] * NBUF
      + [pltpu.VMEM_SHARED((ACC_ROWS, 128), jnp.float32)]
      + [pltpu.SemaphoreType.DMA] * (2 * NBUF),
  )
  return kern(g0, g1, srcp, dstp)


def _spmm48_body(g, srcp, dstp, out, src2d, dst2d, b0, b1,
                 acc, g0s, g1s, s0s, s1s):
  c = lax.axis_index("c")
  s = lax.axis_index("s")
  bufs = (b0, b1)
  gsems = (g0s, g1s)
  ssems = (s0s, s1s)

  _row_copy(s, lambda o, sz: g.at[pl.ds(o, sz)],
            lambda o, sz: acc.at[pl.ds(o, sz)])
  plsc.subcore_barrier()

  chunks_per_tile = NCHUNKS // (NC * NS)
  wid = c * NS + s
  row0 = wid * chunks_per_tile

  def load_idx(p):
    pltpu.sync_copy(srcp.at[pl.ds(row0 + p * IDXB, IDXB)], src2d)
    pltpu.sync_copy(dstp.at[pl.ds(row0 + p * IDXB, IDXB)], dst2d)

  def gather_issue(j, b):
    pltpu.async_copy(g.at[src2d.at[j]], bufs[b], gsems[b])

  def gather_wait(j, b):
    pltpu.make_async_copy(g.at[src2d.at[j]], bufs[b], gsems[b]).wait()

  def scatter_issue(j, b):
    pltpu.async_copy(bufs[b], acc.at[dst2d.at[j]], ssems[b], add=True)

  def scatter_wait(j, b):
    pltpu.make_async_copy(bufs[b], acc.at[dst2d.at[j]], ssems[b]).wait()

  _edge_pipeline(load_idx, gather_issue, gather_wait, scatter_issue,
                 scatter_wait, chunks_per_tile // IDXB)

  plsc.subcore_barrier()
  for ci in (0, 1):
    @pl.when(c == ci)
    def _(ci=ci):
      _row_copy(s, lambda o, sz: acc.at[pl.ds(o, sz)],
                lambda o, sz: out.at[ci, pl.ds(o, sz)])


def _sc_spmm48(g, srcp, dstp):
  kern = pl.kernel(
      _spmm48_body,
      out_type=jax.ShapeDtypeStruct((NC, N_FIX, CP), jnp.float32),
      mesh=_sc_mesh(),
      compiler_params=pltpu.CompilerParams(needs_layout_passes=False,
                                           use_tc_tiling_on_sc=False),
      scratch_types=[
          pltpu.VMEM((IDXB, CHUNK), jnp.int32),
          pltpu.VMEM((IDXB, CHUNK), jnp.int32),
      ] + [pltpu.VMEM((CHUNK, CP), jnp.float32)] * NBUF
      + [pltpu.VMEM_SHARED((ACC_ROWS, CP), jnp.float32)]
      + [pltpu.SemaphoreType.DMA] * (2 * NBUF),
  )
  return kern(g, srcp, dstp)


RB = 1000
NB = N_FIX // RB


def _prep_body(degp_ref, dis_ref):
  d = jnp.sum(degp_ref[...], axis=0) + 1.0
  dis_ref[...] = lax.rsqrt(d)


def _tc_prep(degp):
  return pl.pallas_call(
      _prep_body,
      out_shape=jax.ShapeDtypeStruct((80, 128), jnp.float32),
  )(degp)


def _mm_scale_body(x_ref, w_ref, dis_ref, o_ref):
  acc = jnp.dot(x_ref[...], w_ref[...], preferred_element_type=jnp.float32)
  o_ref[...] = (dis_ref[...] * acc)[None]


def _tc_mm_scale(x, w, dis):
  d_in = x.shape[1]
  return pl.pallas_call(
      _mm_scale_body,
      grid=(NC, NB),
      in_specs=[
          pl.BlockSpec((RB, d_in), lambda c, n: (n, 0)),
          pl.BlockSpec((d_in, 128), lambda c, n: (0, c)),
          pl.BlockSpec((RB, 1), lambda c, n: (n, 0)),
      ],
      out_specs=pl.BlockSpec((1, RB, 128), lambda c, n: (c, n, 0)),
      out_shape=jax.ShapeDtypeStruct((NC, N_FIX, 128), jnp.float32),
  )(x, w, dis)


def _stats_body(s_ref, dis_ref, o_ref, acc):
  n = pl.program_id(0)
  @pl.when(n == 0)
  def _():
    acc[...] = jnp.zeros_like(acc)
  t = jnp.concatenate([s_ref[0], s_ref[1]], axis=1) * dis_ref[...]
  acc[0, :] += jnp.sum(t, axis=0)
  acc[1, :] += jnp.sum(t * t, axis=0)
  @pl.when(n == NB - 1)
  def _():
    o_ref[...] = acc[...] * (1.0 / N_FIX)


def _tc_stats(s, dis):
  return pl.pallas_call(
      _stats_body,
      grid=(NB,),
      in_specs=[
          pl.BlockSpec((NC, RB, 128), lambda n: (0, n, 0)),
          pl.BlockSpec((RB, 1), lambda n: (n, 0)),
      ],
      out_specs=pl.BlockSpec((2, 256), lambda n: (0, 0)),
      out_shape=jax.ShapeDtypeStruct((2, 256), jnp.float32),
      scratch_shapes=[pltpu.VMEM((2, 256), jnp.float32)],
  )(s, dis)


def _bn_mm_body(s_ref, dis_ref, mom_ref, gam_ref, bet_ref, w_ref, o_ref,
                *, ncols):
  t = jnp.concatenate([s_ref[0], s_ref[1]], axis=1) * dis_ref[...]
  m = mom_ref[0, :]
  var = jnp.maximum(mom_ref[1, :] - m * m, 0.0)
  scale = gam_ref[...] * lax.rsqrt(var + 1e-5)
  h = jnp.maximum((t - m) * scale + bet_ref[...], 0.0)
  acc = jnp.dot(h, w_ref[...], preferred_element_type=jnp.float32)
  out = dis_ref[...] * acc
  if o_ref.ndim == 3:
    o_ref[...] = out[None]
  else:
    o_ref[...] = out


def _tc_bn_mm(s, dis, mom, gamma, beta, w):
  return pl.pallas_call(
      functools.partial(_bn_mm_body, ncols=128),
      grid=(NC, NB),
      in_specs=[
          pl.BlockSpec((NC, RB, 128), lambda c, n: (0, n, 0)),
          pl.BlockSpec((RB, 1), lambda c, n: (n, 0)),
          pl.BlockSpec((2, 256), lambda c, n: (0, 0)),
          pl.BlockSpec((1, 256), lambda c, n: (0, 0)),
          pl.BlockSpec((1, 256), lambda c, n: (0, 0)),
          pl.BlockSpec((256, 128), lambda c, n: (0, c)),
      ],
      out_specs=pl.BlockSpec((1, RB, 128), lambda c, n: (c, n, 0)),
      out_shape=jax.ShapeDtypeStruct((NC, N_FIX, 128), jnp.float32),
  )(s, dis, mom, gamma, beta, w)


def _tc_bn_mm3(s, dis, mom, gamma, beta, w):
  return pl.pallas_call(
      functools.partial(_bn_mm_body, ncols=CP),
      grid=(NB,),
      in_specs=[
          pl.BlockSpec((NC, RB, 128), lambda n: (0, n, 0)),
          pl.BlockSpec((RB, 1), lambda n: (n, 0)),
          pl.BlockSpec((2, 256), lambda n: (0, 0)),
          pl.BlockSpec((1, 256), lambda n: (0, 0)),
          pl.BlockSpec((1, 256), lambda n: (0, 0)),
          pl.BlockSpec((256, CP), lambda n: (0, 0)),
      ],
      out_specs=pl.BlockSpec((RB, CP), lambda n: (n, 0)),
      out_shape=jax.ShapeDtypeStruct((N_FIX, CP), jnp.float32),
  )(s, dis, mom, gamma, beta, w)


def _final_body(s_ref, g_ref, dis_ref, b_ref, o_ref):
  z = dis_ref[...] * (s_ref[0] + s_ref[1] - g_ref[...]) + b_ref[...]
  z40 = z[:, :40]
  m = jnp.max(z40, axis=1, keepdims=True)
  e = jnp.exp(z40 - m)
  lse = jnp.log(jnp.sum(e, axis=1, keepdims=True))
  o_ref[...] = z40 - m - lse


def _tc_final(s3, g3, dis, b3p):
  return pl.pallas_call(
      _final_body,
      grid=(NB,),
      in_specs=[
          pl.BlockSpec((NC, RB, CP), lambda n: (0, n, 0)),
          pl.BlockSpec((RB, CP), lambda n: (n, 0)),
          pl.BlockSpec((RB, 1), lambda n: (n, 0)),
          pl.BlockSpec((1, CP), lambda n: (0, 0)),
      ],
      out_specs=pl.BlockSpec((RB, 40), lambda n: (n, 0)),
      out_shape=jax.ShapeDtypeStruct((N_FIX, 40), jnp.float32),
  )(s3, g3, dis, b3p)


@jax.jit
def _run(x, edge_index, W1, b1, gamma1, beta1, W2, b2, gamma2, beta2, W3, b3):
  n = x.shape[0]
  e = edge_index.shape[1]
  pad = E_PAD - e
  srcp = jnp.concatenate(
      [edge_index[0], jnp.zeros((pad,), jnp.int32)]).reshape(NCHUNKS, CHUNK)
  dstp = jnp.concatenate(
      [edge_index[1], jnp.full((pad,), n, jnp.int32)]).reshape(NCHUNKS, CHUNK)

  degp = _sc_degree(dstp).reshape(NC * NS, 80, 128)
  dis = _tc_prep(degp).reshape(-1)[:n].reshape(n, 1)

  gamma1r = gamma1.reshape(1, -1)
  beta1r = beta1.reshape(1, -1)
  gamma2r = gamma2.reshape(1, -1)
  beta2r = beta2.reshape(1, -1)
  w3p = jnp.pad(W3, ((0, 0), (0, CP - W3.shape[1])))
  b3p = jnp.pad(b3, (0, CP - b3.shape[0])).reshape(1, CP)

  g1 = _tc_mm_scale(x, W1, dis)
  s1 = _sc_spmm256(g1[0], g1[1], srcp, dstp)
  mom1 = _tc_stats(s1, dis)
  g2 = _tc_bn_mm(s1, dis, mom1, gamma1r, beta1r, W2)
  s2 = _sc_spmm256(g2[0], g2[1], srcp, dstp)
  mom2 = _tc_stats(s2, dis)
  g3 = _tc_bn_mm3(s2, dis, mom2, gamma2r, beta2r, w3p)
  s3 = _sc_spmm48(g3, srcp, dstp)
  return _tc_final(s3, g3, dis, b3p)


def kernel(x, edge_index, W1, b1, gamma1, beta1, W2, b2, gamma2, beta2,
           W3, b3):
  return _run(x, edge_index, W1, b1, gamma1, beta1, W2, b2, gamma2, beta2,
              W3, b3)


# speedup vs baseline: 1.0391x; 1.0391x over previous
import functools

import jax
import jax.numpy as jnp
from jax import lax
from jax.experimental import pallas as pl
from jax.experimental.pallas import tpu as pltpu
from jax.experimental.pallas import tpu_sc as plsc

NC, NS, L = 2, 16, 16
CHUNK = 160
N_FIX = 10000
E_FIX = 160000
E_PAD = 163840
NCHUNKS = E_PAD // CHUNK
ROWS_PER_TILE = N_FIX // NS
ACC_ROWS = N_FIX + 8
CP = 128


def _sc_mesh():
  return plsc.VectorSubcoreMesh(core_axis_name="c", subcore_axis_name="s")


def _tile_rows(s):
  r0 = pl.multiple_of(s * 624, 8)
  return r0


def _row_copy(s, mk_src, mk_dst):
  @pl.when(s < NS - 1)
  def _():
    r0 = _tile_rows(s)
    pltpu.sync_copy(mk_src(r0, 624), mk_dst(r0, 624))
  @pl.when(s == NS - 1)
  def _():
    pltpu.sync_copy(mk_src(624 * (NS - 1), 640), mk_dst(624 * (NS - 1), 640))


def _deg_body(dstp, out, idx_v, hist, sem):
  c = lax.axis_index("c")
  s = lax.axis_index("s")
  wid = c * NS + s
  chunks_per_tile = NCHUNKS // (NC * NS)

  def zero_body(i, carry):
    hist[pl.ds(i * L, L)] = jnp.zeros((L,), jnp.float32)
    return carry
  lax.fori_loop(0, 10240 // L, zero_body, 0)

  ones = jnp.ones((L,), jnp.float32)

  def body(j, carry):
    pltpu.sync_copy(dstp.at[wid * chunks_per_tile + j], idx_v)
    for k in range(CHUNK // L):
      idx = idx_v[pl.ds(k * L, L)]
      plsc.addupdate_scatter(hist, [idx], ones)
    return carry
  lax.fori_loop(0, chunks_per_tile, body, 0)

  pltpu.sync_copy(hist, out.at[wid])


def _sc_degree(dstp):
  kern = pl.kernel(
      _deg_body,
      out_type=jax.ShapeDtypeStruct((NC * NS, 10240), jnp.float32),
      mesh=_sc_mesh(),
      compiler_params=pltpu.CompilerParams(needs_layout_passes=False),
      scratch_types=[
          pltpu.VMEM((CHUNK,), jnp.int32),
          pltpu.VMEM((10240,), jnp.float32),
          pltpu.SemaphoreType.DMA,
      ],
  )
  return kern(dstp)


NBUF = 2
IDXB = 16


def _edge_pipeline(load_idx, gather_issue, gather_wait, scatter, nphases):
  for p in range(nphases):
    load_idx(p)
    for b in range(NBUF):
      gather_issue(b, b)

    def body(t, carry):
      for b in range(NBUF):
        j = t * NBUF + b
        gather_wait(j, b)
        scatter(j, b)
        @pl.when(j + NBUF < IDXB)
        def _(j=j, b=b):
          gather_issue(j + NBUF, b)
      return carry
    lax.fori_loop(0, IDXB // NBUF, body, 0)


def _spmm_body(g0, g1, srcp, dstp, out, src2d, dst2d, b0, b1,
               acc, g0s, g1s):
  c = lax.axis_index("c")
  s = lax.axis_index("s")
  bufs = (b0, b1)
  gsems = (g0s, g1s)

  chunks_per_tile = NCHUNKS // NS
  for ci in (0, 1):
    @pl.when(c == ci)
    def _(ci=ci):
      g_c = (g0, g1)[ci]
      _row_copy(s, lambda o, sz: g_c.at[pl.ds(o, sz)],
                lambda o, sz: acc.at[pl.ds(o, sz)])
  plsc.subcore_barrier()

  row0 = s * chunks_per_tile

  def load_idx(p):
    pltpu.sync_copy(srcp.at[pl.ds(row0 + p * IDXB, IDXB)], src2d)
    pltpu.sync_copy(dstp.at[pl.ds(row0 + p * IDXB, IDXB)], dst2d)

  def gather_issue(j, b):
    for ci in (0, 1):
      @pl.when(c == ci)
      def _(ci=ci):
        pltpu.async_copy((g0, g1)[ci].at[src2d.at[j]], bufs[b], gsems[b])

  def gather_wait(j, b):
    for ci in (0, 1):
      @pl.when(c == ci)
      def _(ci=ci):
        pltpu.make_async_copy((g0, g1)[ci].at[src2d.at[j]], bufs[b],
                              gsems[b]).wait()

  def scatter(j, b):
    pltpu.sync_copy(bufs[b], acc.at[dst2d.at[j]], add=True)

  _edge_pipeline(load_idx, gather_issue, gather_wait, scatter,
                 chunks_per_tile // IDXB)

  plsc.subcore_barrier()
  for ci in (0, 1):
    @pl.when(c == ci)
    def _(ci=ci):
      _row_copy(s, lambda o, sz: acc.at[pl.ds(o, sz)],
                lambda o, sz: out.at[ci, pl.ds(o, sz)])


def _sc_spmm256(g0, g1, srcp, dstp):
  kern = pl.kernel(
      _spmm_body,
      out_type=jax.ShapeDtypeStruct((NC, N_FIX, 128), jnp.float32),
      mesh=_sc_mesh(),
      compiler_params=pltpu.CompilerParams(needs_layout_passes=False,
                                           use_tc_tiling_on_sc=False),
      scratch_types=[
          pltpu.VMEM((IDXB, CHUNK), jnp.int32),
          pltpu.VMEM((IDXB, CHUNK), jnp.int32),
      ] + [pltpu.VMEM((CHUNK, 128), jnp.float32)] * NBUF
      + [pltpu.VMEM_SHARED((ACC_ROWS, 128), jnp.float32)]
      + [pltpu.SemaphoreType.DMA] * NBUF,
  )
  return kern(g0, g1, srcp, dstp)


def _spmm48_body(g, srcp, dstp, out, src2d, dst2d, b0, b1,
                 acc, g0s, g1s):
  c = lax.axis_index("c")
  s = lax.axis_index("s")
  bufs = (b0, b1)
  gsems = (g0s, g1s)

  _row_copy(s, lambda o, sz: g.at[pl.ds(o, sz)],
            lambda o, sz: acc.at[pl.ds(o, sz)])
  plsc.subcore_barrier()

  chunks_per_tile = NCHUNKS // (NC * NS)
  wid = c * NS + s
  row0 = wid * chunks_per_tile

  def load_idx(p):
    pltpu.sync_copy(srcp.at[pl.ds(row0 + p * IDXB, IDXB)], src2d)
    pltpu.sync_copy(dstp.at[pl.ds(row0 + p * IDXB, IDXB)], dst2d)

  def gather_issue(j, b):
    pltpu.async_copy(g.at[src2d.at[j]], bufs[b], gsems[b])

  def gather_wait(j, b):
    pltpu.make_async_copy(g.at[src2d.at[j]], bufs[b], gsems[b]).wait()

  def scatter(j, b):
    pltpu.sync_copy(bufs[b], acc.at[dst2d.at[j]], add=True)

  _edge_pipeline(load_idx, gather_issue, gather_wait, scatter,
                 chunks_per_tile // IDXB)

  plsc.subcore_barrier()
  for ci in (0, 1):
    @pl.when(c == ci)
    def _(ci=ci):
      _row_copy(s, lambda o, sz: acc.at[pl.ds(o, sz)],
                lambda o, sz: out.at[ci, pl.ds(o, sz)])


def _sc_spmm48(g, srcp, dstp):
  kern = pl.kernel(
      _spmm48_body,
      out_type=jax.ShapeDtypeStruct((NC, N_FIX, CP), jnp.float32),
      mesh=_sc_mesh(),
      compiler_params=pltpu.CompilerParams(needs_layout_passes=False,
                                           use_tc_tiling_on_sc=False),
      scratch_types=[
          pltpu.VMEM((IDXB, CHUNK), jnp.int32),
          pltpu.VMEM((IDXB, CHUNK), jnp.int32),
      ] + [pltpu.VMEM((CHUNK, CP), jnp.float32)] * NBUF
      + [pltpu.VMEM_SHARED((ACC_ROWS, CP), jnp.float32)]
      + [pltpu.SemaphoreType.DMA] * NBUF,
  )
  return kern(g, srcp, dstp)


RB = 1000
NB = N_FIX // RB


def _prep_body(degp_ref, dis_ref):
  d = jnp.sum(degp_ref[...], axis=0) + 1.0
  dis_ref[...] = lax.rsqrt(d)


def _tc_prep(degp):
  return pl.pallas_call(
      _prep_body,
      out_shape=jax.ShapeDtypeStruct((80, 128), jnp.float32),
  )(degp)


def _mm_scale_body(x_ref, w_ref, dis_ref, o_ref):
  acc = jnp.dot(x_ref[...], w_ref[...], preferred_element_type=jnp.float32)
  o_ref[...] = (dis_ref[...] * acc)[None]


def _tc_mm_scale(x, w, dis):
  d_in = x.shape[1]
  return pl.pallas_call(
      _mm_scale_body,
      grid=(NC, NB),
      in_specs=[
          pl.BlockSpec((RB, d_in), lambda c, n: (n, 0)),
          pl.BlockSpec((d_in, 128), lambda c, n: (0, c)),
          pl.BlockSpec((RB, 1), lambda c, n: (n, 0)),
      ],
      out_specs=pl.BlockSpec((1, RB, 128), lambda c, n: (c, n, 0)),
      out_shape=jax.ShapeDtypeStruct((NC, N_FIX, 128), jnp.float32),
  )(x, w, dis)


def _stats_body(s_ref, dis_ref, o_ref, acc):
  n = pl.program_id(0)
  @pl.when(n == 0)
  def _():
    acc[...] = jnp.zeros_like(acc)
  t = jnp.concatenate([s_ref[0], s_ref[1]], axis=1) * dis_ref[...]
  acc[0, :] += jnp.sum(t, axis=0)
  acc[1, :] += jnp.sum(t * t, axis=0)
  @pl.when(n == NB - 1)
  def _():
    o_ref[...] = acc[...] * (1.0 / N_FIX)


def _tc_stats(s, dis):
  return pl.pallas_call(
      _stats_body,
      grid=(NB,),
      in_specs=[
          pl.BlockSpec((NC, RB, 128), lambda n: (0, n, 0)),
          pl.BlockSpec((RB, 1), lambda n: (n, 0)),
      ],
      out_specs=pl.BlockSpec((2, 256), lambda n: (0, 0)),
      out_shape=jax.ShapeDtypeStruct((2, 256), jnp.float32),
      scratch_shapes=[pltpu.VMEM((2, 256), jnp.float32)],
  )(s, dis)


def _bn_mm_body(s_ref, dis_ref, mom_ref, gam_ref, bet_ref, w_ref, o_ref,
                *, ncols):
  t = jnp.concatenate([s_ref[0], s_ref[1]], axis=1) * dis_ref[...]
  m = mom_ref[0, :]
  var = jnp.maximum(mom_ref[1, :] - m * m, 0.0)
  scale = gam_ref[...] * lax.rsqrt(var + 1e-5)
  h = jnp.maximum((t - m) * scale + bet_ref[...], 0.0)
  acc = jnp.dot(h, w_ref[...], preferred_element_type=jnp.float32)
  out = dis_ref[...] * acc
  if o_ref.ndim == 3:
    o_ref[...] = out[None]
  else:
    o_ref[...] = out


def _tc_bn_mm(s, dis, mom, gamma, beta, w):
  return pl.pallas_call(
      functools.partial(_bn_mm_body, ncols=128),
      grid=(NC, NB),
      in_specs=[
          pl.BlockSpec((NC, RB, 128), lambda c, n: (0, n, 0)),
          pl.BlockSpec((RB, 1), lambda c, n: (n, 0)),
          pl.BlockSpec((2, 256), lambda c, n: (0, 0)),
          pl.BlockSpec((1, 256), lambda c, n: (0, 0)),
          pl.BlockSpec((1, 256), lambda c, n: (0, 0)),
          pl.BlockSpec((256, 128), lambda c, n: (0, c)),
      ],
      out_specs=pl.BlockSpec((1, RB, 128), lambda c, n: (c, n, 0)),
      out_shape=jax.ShapeDtypeStruct((NC, N_FIX, 128), jnp.float32),
  )(s, dis, mom, gamma, beta, w)


def _tc_bn_mm3(s, dis, mom, gamma, beta, w):
  return pl.pallas_call(
      functools.partial(_bn_mm_body, ncols=CP),
      grid=(NB,),
      in_specs=[
          pl.BlockSpec((NC, RB, 128), lambda n: (0, n, 0)),
          pl.BlockSpec((RB, 1), lambda n: (n, 0)),
          pl.BlockSpec((2, 256), lambda n: (0, 0)),
          pl.BlockSpec((1, 256), lambda n: (0, 0)),
          pl.BlockSpec((1, 256), lambda n: (0, 0)),
          pl.BlockSpec((256, CP), lambda n: (0, 0)),
      ],
      out_specs=pl.BlockSpec((RB, CP), lambda n: (n, 0)),
      out_shape=jax.ShapeDtypeStruct((N_FIX, CP), jnp.float32),
  )(s, dis, mom, gamma, beta, w)


def _final_body(s_ref, g_ref, dis_ref, b_ref, o_ref):
  z = dis_ref[...] * (s_ref[0] + s_ref[1] - g_ref[...]) + b_ref[...]
  z40 = z[:, :40]
  m = jnp.max(z40, axis=1, keepdims=True)
  e = jnp.exp(z40 - m)
  lse = jnp.log(jnp.sum(e, axis=1, keepdims=True))
  o_ref[...] = z40 - m - lse


def _tc_final(s3, g3, dis, b3p):
  return pl.pallas_call(
      _final_body,
      grid=(NB,),
      in_specs=[
          pl.BlockSpec((NC, RB, CP), lambda n: (0, n, 0)),
          pl.BlockSpec((RB, CP), lambda n: (n, 0)),
          pl.BlockSpec((RB, 1), lambda n: (n, 0)),
          pl.BlockSpec((1, CP), lambda n: (0, 0)),
      ],
      out_specs=pl.BlockSpec((RB, 40), lambda n: (n, 0)),
      out_shape=jax.ShapeDtypeStruct((N_FIX, 40), jnp.float32),
  )(s3, g3, dis, b3p)


@jax.jit
def _run(x, edge_index, W1, b1, gamma1, beta1, W2, b2, gamma2, beta2, W3, b3):
  n = x.shape[0]
  e = edge_index.shape[1]
  pad = E_PAD - e
  srcp = jnp.concatenate(
      [edge_index[0], jnp.zeros((pad,), jnp.int32)]).reshape(NCHUNKS, CHUNK)
  dstp = jnp.concatenate(
      [edge_index[1], jnp.full((pad,), n, jnp.int32)]).reshape(NCHUNKS, CHUNK)

  degp = _sc_degree(dstp).reshape(NC * NS, 80, 128)
  dis = _tc_prep(degp).reshape(-1)[:n].reshape(n, 1)

  gamma1r = gamma1.reshape(1, -1)
  beta1r = beta1.reshape(1, -1)
  gamma2r = gamma2.reshape(1, -1)
  beta2r = beta2.reshape(1, -1)
  w3p = jnp.pad(W3, ((0, 0), (0, CP - W3.shape[1])))
  b3p = jnp.pad(b3, (0, CP - b3.shape[0])).reshape(1, CP)

  g1 = _tc_mm_scale(x, W1, dis)
  s1 = _sc_spmm256(g1[0], g1[1], srcp, dstp)
  mom1 = _tc_stats(s1, dis)
  g2 = _tc_bn_mm(s1, dis, mom1, gamma1r, beta1r, W2)
  s2 = _sc_spmm256(g2[0], g2[1], srcp, dstp)
  mom2 = _tc_stats(s2, dis)
  g3 = _tc_bn_mm3(s2, dis, mom2, gamma2r, beta2r, w3p)
  s3 = _sc_spmm48(g3, srcp, dstp)
  return _tc_final(s3, g3, dis, b3p)


def kernel(x, edge_index, W1, b1, gamma1, beta1, W2, b2, gamma2, beta2,
           W3, b3):
  return _run(x, edge_index, W1, b1, gamma1, beta1, W2, b2, gamma2, beta2,
              W3, b3)


# speedup vs baseline: 1.1384x; 1.0956x over previous
import functools

import jax
import jax.numpy as jnp
from jax import lax
from jax.experimental import pallas as pl
from jax.experimental.pallas import tpu as pltpu
from jax.experimental.pallas import tpu_sc as plsc

NC, NS, L = 2, 16, 16
CHUNK = 128
N_FIX = 10000
E_FIX = 160000
E_PAD = 163840
NCHUNKS = E_PAD // CHUNK
ROWS_PER_TILE = N_FIX // NS
ACC_ROWS = N_FIX + 8
CP = 128


def _sc_mesh():
  return plsc.VectorSubcoreMesh(core_axis_name="c", subcore_axis_name="s")


def _tile_rows(s):
  r0 = pl.multiple_of(s * 624, 8)
  return r0


def _row_copy(s, mk_src, mk_dst):
  @pl.when(s < NS - 1)
  def _():
    r0 = _tile_rows(s)
    pltpu.sync_copy(mk_src(r0, 624), mk_dst(r0, 624))
  @pl.when(s == NS - 1)
  def _():
    pltpu.sync_copy(mk_src(624 * (NS - 1), 640), mk_dst(624 * (NS - 1), 640))


def _deg_body(dstp, out, idx_v, hist, sem):
  c = lax.axis_index("c")
  s = lax.axis_index("s")
  wid = c * NS + s
  chunks_per_tile = NCHUNKS // (NC * NS)

  def zero_body(i, carry):
    hist[pl.ds(i * L, L)] = jnp.zeros((L,), jnp.float32)
    return carry
  lax.fori_loop(0, 10240 // L, zero_body, 0)

  ones = jnp.ones((L,), jnp.float32)

  def body(j, carry):
    pltpu.sync_copy(dstp.at[wid * chunks_per_tile + j], idx_v)
    for k in range(CHUNK // L):
      idx = idx_v[pl.ds(k * L, L)]
      plsc.addupdate_scatter(hist, [idx], ones)
    return carry
  lax.fori_loop(0, chunks_per_tile, body, 0)

  pltpu.sync_copy(hist, out.at[wid])


def _sc_degree(dstp):
  kern = pl.kernel(
      _deg_body,
      out_type=jax.ShapeDtypeStruct((NC * NS, 10240), jnp.float32),
      mesh=_sc_mesh(),
      compiler_params=pltpu.CompilerParams(needs_layout_passes=False),
      scratch_types=[
          pltpu.VMEM((CHUNK,), jnp.int32),
          pltpu.VMEM((10240,), jnp.float32),
          pltpu.SemaphoreType.DMA,
      ],
  )
  return kern(dstp)


NBUF = 2
IDXB = 40


def _edge_pipeline(load_idx, gather_issue, gather_wait, scatter, nphases):
  for p in range(nphases):
    load_idx(p)
    for b in range(NBUF):
      gather_issue(b, b)

    def body(t, carry):
      for b in range(NBUF):
        j = t * NBUF + b
        gather_wait(j, b)
        scatter(j, b)
        @pl.when(j + NBUF < IDXB)
        def _(j=j, b=b):
          gather_issue(j + NBUF, b)
      return carry
    lax.fori_loop(0, IDXB // NBUF, body, 0)


def _spmm_body(g0, g1, srcp, dstp, out, src2d, dst2d, b0, b1,
               acc, g0s, g1s):
  c = lax.axis_index("c")
  s = lax.axis_index("s")
  bufs = (b0, b1)
  gsems = (g0s, g1s)

  chunks_per_tile = NCHUNKS // NS
  for ci in (0, 1):
    @pl.when(c == ci)
    def _(ci=ci):
      g_c = (g0, g1)[ci]
      _row_copy(s, lambda o, sz: g_c.at[pl.ds(o, sz)],
                lambda o, sz: acc.at[pl.ds(o, sz)])
  plsc.subcore_barrier()

  row0 = s * chunks_per_tile

  def load_idx(p):
    pltpu.sync_copy(srcp.at[pl.ds(row0 + p * IDXB, IDXB)], src2d)
    pltpu.sync_copy(dstp.at[pl.ds(row0 + p * IDXB, IDXB)], dst2d)

  def gather_issue(j, b):
    for ci in (0, 1):
      @pl.when(c == ci)
      def _(ci=ci):
        pltpu.async_copy((g0, g1)[ci].at[src2d.at[j]], bufs[b], gsems[b])

  def gather_wait(j, b):
    for ci in (0, 1):
      @pl.when(c == ci)
      def _(ci=ci):
        pltpu.make_async_copy((g0, g1)[ci].at[src2d.at[j]], bufs[b],
                              gsems[b]).wait()

  def scatter(j, b):
    pltpu.sync_copy(bufs[b], acc.at[dst2d.at[j]], add=True)

  _edge_pipeline(load_idx, gather_issue, gather_wait, scatter,
                 chunks_per_tile // IDXB)

  plsc.subcore_barrier()
  for ci in (0, 1):
    @pl.when(c == ci)
    def _(ci=ci):
      _row_copy(s, lambda o, sz: acc.at[pl.ds(o, sz)],
                lambda o, sz: out.at[ci, pl.ds(o, sz)])


def _sc_spmm256(g0, g1, srcp, dstp):
  kern = pl.kernel(
      _spmm_body,
      out_type=jax.ShapeDtypeStruct((NC, N_FIX, 128), jnp.float32),
      mesh=_sc_mesh(),
      compiler_params=pltpu.CompilerParams(needs_layout_passes=False,
                                           use_tc_tiling_on_sc=False),
      scratch_types=[
          pltpu.VMEM((IDXB, CHUNK), jnp.int32),
          pltpu.VMEM((IDXB, CHUNK), jnp.int32),
      ] + [pltpu.VMEM((CHUNK, 128), jnp.float32)] * NBUF
      + [pltpu.VMEM_SHARED((ACC_ROWS, 128), jnp.float32)]
      + [pltpu.SemaphoreType.DMA] * NBUF,
  )
  return kern(g0, g1, srcp, dstp)


def _spmm48_body(g, srcp, dstp, out, src2d, dst2d, b0, b1,
                 acc, g0s, g1s):
  c = lax.axis_index("c")
  s = lax.axis_index("s")
  bufs = (b0, b1)
  gsems = (g0s, g1s)

  _row_copy(s, lambda o, sz: g.at[pl.ds(o, sz)],
            lambda o, sz: acc.at[pl.ds(o, sz)])
  plsc.subcore_barrier()

  chunks_per_tile = NCHUNKS // (NC * NS)
  wid = c * NS + s
  row0 = wid * chunks_per_tile

  def load_idx(p):
    pltpu.sync_copy(srcp.at[pl.ds(row0 + p * IDXB, IDXB)], src2d)
    pltpu.sync_copy(dstp.at[pl.ds(row0 + p * IDXB, IDXB)], dst2d)

  def gather_issue(j, b):
    pltpu.async_copy(g.at[src2d.at[j]], bufs[b], gsems[b])

  def gather_wait(j, b):
    pltpu.make_async_copy(g.at[src2d.at[j]], bufs[b], gsems[b]).wait()

  def scatter(j, b):
    pltpu.sync_copy(bufs[b], acc.at[dst2d.at[j]], add=True)

  _edge_pipeline(load_idx, gather_issue, gather_wait, scatter,
                 chunks_per_tile // IDXB)

  plsc.subcore_barrier()
  for ci in (0, 1):
    @pl.when(c == ci)
    def _(ci=ci):
      _row_copy(s, lambda o, sz: acc.at[pl.ds(o, sz)],
                lambda o, sz: out.at[ci, pl.ds(o, sz)])


def _sc_spmm48(g, srcp, dstp):
  kern = pl.kernel(
      _spmm48_body,
      out_type=jax.ShapeDtypeStruct((NC, N_FIX, CP), jnp.float32),
      mesh=_sc_mesh(),
      compiler_params=pltpu.CompilerParams(needs_layout_passes=False,
                                           use_tc_tiling_on_sc=False),
      scratch_types=[
          pltpu.VMEM((IDXB, CHUNK), jnp.int32),
          pltpu.VMEM((IDXB, CHUNK), jnp.int32),
      ] + [pltpu.VMEM((CHUNK, CP), jnp.float32)] * NBUF
      + [pltpu.VMEM_SHARED((ACC_ROWS, CP), jnp.float32)]
      + [pltpu.SemaphoreType.DMA] * NBUF,
  )
  return kern(g, srcp, dstp)


RB = 1000
NB = N_FIX // RB


def _prep_body(degp_ref, dis_ref):
  d = jnp.sum(degp_ref[...], axis=0) + 1.0
  dis_ref[...] = lax.rsqrt(d)


def _tc_prep(degp):
  return pl.pallas_call(
      _prep_body,
      out_shape=jax.ShapeDtypeStruct((80, 128), jnp.float32),
  )(degp)


def _mm_body(x_ref, w_ref, o_ref):
  acc = jnp.dot(x_ref[...], w_ref[...], preferred_element_type=jnp.float32)
  o_ref[...] = acc[None]


def _tc_mm(x, w):
  d_in = x.shape[1]
  return pl.pallas_call(
      _mm_body,
      grid=(NC, NB),
      in_specs=[
          pl.BlockSpec((RB, d_in), lambda c, n: (n, 0)),
          pl.BlockSpec((d_in, 128), lambda c, n: (0, c)),
      ],
      out_specs=pl.BlockSpec((1, RB, 128), lambda c, n: (c, n, 0)),
      out_shape=jax.ShapeDtypeStruct((NC, N_FIX, 128), jnp.float32),
  )(x, w)


def _scale_body(xw_ref, dis_ref, o_ref):
  o_ref[...] = dis_ref[...][None] * xw_ref[...]


def _tc_scale(xw, dis):
  return pl.pallas_call(
      _scale_body,
      grid=(NB,),
      in_specs=[
          pl.BlockSpec((NC, RB, 128), lambda n: (0, n, 0)),
          pl.BlockSpec((RB, 1), lambda n: (n, 0)),
      ],
      out_specs=pl.BlockSpec((NC, RB, 128), lambda n: (0, n, 0)),
      out_shape=jax.ShapeDtypeStruct((NC, N_FIX, 128), jnp.float32),
  )(xw, dis)


def _stats_body(s_ref, dis_ref, o_ref, acc):
  n = pl.program_id(0)
  @pl.when(n == 0)
  def _():
    acc[...] = jnp.zeros_like(acc)
  t = jnp.concatenate([s_ref[0], s_ref[1]], axis=1) * dis_ref[...]
  acc[0, :] += jnp.sum(t, axis=0)
  acc[1, :] += jnp.sum(t * t, axis=0)
  @pl.when(n == NB - 1)
  def _():
    o_ref[...] = acc[...] * (1.0 / N_FIX)


def _tc_stats(s, dis):
  return pl.pallas_call(
      _stats_body,
      grid=(NB,),
      in_specs=[
          pl.BlockSpec((NC, RB, 128), lambda n: (0, n, 0)),
          pl.BlockSpec((RB, 1), lambda n: (n, 0)),
      ],
      out_specs=pl.BlockSpec((2, 256), lambda n: (0, 0)),
      out_shape=jax.ShapeDtypeStruct((2, 256), jnp.float32),
      scratch_shapes=[pltpu.VMEM((2, 256), jnp.float32)],
  )(s, dis)


def _bn_mm_body(s_ref, dis_ref, mom_ref, gam_ref, bet_ref, w_ref, o_ref,
                *, ncols):
  t = jnp.concatenate([s_ref[0], s_ref[1]], axis=1) * dis_ref[...]
  m = mom_ref[0, :]
  var = jnp.maximum(mom_ref[1, :] - m * m, 0.0)
  scale = gam_ref[...] * lax.rsqrt(var + 1e-5)
  h = jnp.maximum((t - m) * scale + bet_ref[...], 0.0)
  acc = jnp.dot(h, w_ref[...], preferred_element_type=jnp.float32)
  out = dis_ref[...] * acc
  if o_ref.ndim == 3:
    o_ref[...] = out[None]
  else:
    o_ref[...] = out


def _tc_bn_mm(s, dis, mom, gamma, beta, w):
  return pl.pallas_call(
      functools.partial(_bn_mm_body, ncols=128),
      grid=(NC, NB),
      in_specs=[
          pl.BlockSpec((NC, RB, 128), lambda c, n: (0, n, 0)),
          pl.BlockSpec((RB, 1), lambda c, n: (n, 0)),
          pl.BlockSpec((2, 256), lambda c, n: (0, 0)),
          pl.BlockSpec((1, 256), lambda c, n: (0, 0)),
          pl.BlockSpec((1, 256), lambda c, n: (0, 0)),
          pl.BlockSpec((256, 128), lambda c, n: (0, c)),
      ],
      out_specs=pl.BlockSpec((1, RB, 128), lambda c, n: (c, n, 0)),
      out_shape=jax.ShapeDtypeStruct((NC, N_FIX, 128), jnp.float32),
  )(s, dis, mom, gamma, beta, w)


def _tc_bn_mm3(s, dis, mom, gamma, beta, w):
  return pl.pallas_call(
      functools.partial(_bn_mm_body, ncols=CP),
      grid=(NB,),
      in_specs=[
          pl.BlockSpec((NC, RB, 128), lambda n: (0, n, 0)),
          pl.BlockSpec((RB, 1), lambda n: (n, 0)),
          pl.BlockSpec((2, 256), lambda n: (0, 0)),
          pl.BlockSpec((1, 256), lambda n: (0, 0)),
          pl.BlockSpec((1, 256), lambda n: (0, 0)),
          pl.BlockSpec((256, CP), lambda n: (0, 0)),
      ],
      out_specs=pl.BlockSpec((RB, CP), lambda n: (n, 0)),
      out_shape=jax.ShapeDtypeStruct((N_FIX, CP), jnp.float32),
  )(s, dis, mom, gamma, beta, w)


def _final_body(s_ref, g_ref, dis_ref, b_ref, o_ref):
  z = dis_ref[...] * (s_ref[0] + s_ref[1] - g_ref[...]) + b_ref[...]
  z40 = z[:, :40]
  m = jnp.max(z40, axis=1, keepdims=True)
  e = jnp.exp(z40 - m)
  lse = jnp.log(jnp.sum(e, axis=1, keepdims=True))
  o_ref[...] = z40 - m - lse


def _tc_final(s3, g3, dis, b3p):
  return pl.pallas_call(
      _final_body,
      grid=(NB,),
      in_specs=[
          pl.BlockSpec((NC, RB, CP), lambda n: (0, n, 0)),
          pl.BlockSpec((RB, CP), lambda n: (n, 0)),
          pl.BlockSpec((RB, 1), lambda n: (n, 0)),
          pl.BlockSpec((1, CP), lambda n: (0, 0)),
      ],
      out_specs=pl.BlockSpec((RB, 40), lambda n: (n, 0)),
      out_shape=jax.ShapeDtypeStruct((N_FIX, 40), jnp.float32),
  )(s3, g3, dis, b3p)


@jax.jit
def _run(x, edge_index, W1, b1, gamma1, beta1, W2, b2, gamma2, beta2, W3, b3):
  n = x.shape[0]
  e = edge_index.shape[1]
  pad = E_PAD - e
  srcp = jnp.concatenate(
      [edge_index[0], jnp.zeros((pad,), jnp.int32)]).reshape(NCHUNKS, CHUNK)
  dstp = jnp.concatenate(
      [edge_index[1], jnp.full((pad,), n, jnp.int32)]).reshape(NCHUNKS, CHUNK)

  degp = _sc_degree(dstp).reshape(NC * NS, 80, 128)
  dis = _tc_prep(degp).reshape(-1)[:n].reshape(n, 1)

  gamma1r = gamma1.reshape(1, -1)
  beta1r = beta1.reshape(1, -1)
  gamma2r = gamma2.reshape(1, -1)
  beta2r = beta2.reshape(1, -1)
  w3p = jnp.pad(W3, ((0, 0), (0, CP - W3.shape[1])))
  b3p = jnp.pad(b3, (0, CP - b3.shape[0])).reshape(1, CP)

  xw = _tc_mm(x, W1)
  g1 = _tc_scale(xw, dis)
  s1 = _sc_spmm256(g1[0], g1[1], srcp, dstp)
  mom1 = _tc_stats(s1, dis)
  g2 = _tc_bn_mm(s1, dis, mom1, gamma1r, beta1r, W2)
  s2 = _sc_spmm256(g2[0], g2[1], srcp, dstp)
  mom2 = _tc_stats(s2, dis)
  g3 = _tc_bn_mm3(s2, dis, mom2, gamma2r, beta2r, w3p)
  s3 = _sc_spmm48(g3, srcp, dstp)
  return _tc_final(s3, g3, dis, b3p)


def kernel(x, edge_index, W1, b1, gamma1, beta1, W2, b2, gamma2, beta2,
           W3, b3):
  return _run(x, edge_index, W1, b1, gamma1, beta1, W2, b2, gamma2, beta2,
              W3, b3)
